# baseline (speedup 1.0000x reference)
import functools

import jax
import jax.numpy as jnp
from jax import lax
from jax.experimental import pallas as pl
from jax.experimental.pallas import tpu as pltpu
from jax.experimental.pallas import tpu_sc as plsc

_N = 16384
_D = 192
_H = 384
_K = 16
_STRIDE = 4
_Q = _N // _STRIDE
_NV = _D // 3
_BN = 2048
_BQ = 128
_BM = 128
_HIGH = lax.Precision.HIGHEST


def _proj_body(xr, lf9, pos, w1b, w1c, f0, f1, f2, g):
    xb = xr[...]
    rows = lax.broadcasted_iota(jnp.int32, (_D, _NV), 0)
    cols = lax.broadcasted_iota(jnp.int32, (_D, _NV), 1)
    xs = tuple(
        jnp.dot(xb, (rows == 3 * cols + j).astype(jnp.float32),
                precision=_HIGH)
        for j in range(3))
    xsb = tuple(v.astype(jnp.bfloat16).astype(jnp.float32) for v in xs)
    lf = lf9[...].astype(jnp.bfloat16).astype(jnp.float32)
    for i, fo in enumerate((f0, f1, f2)):
        acc = lf[:, i:i + 1] * xsb[0]
        acc = acc + lf[:, 3 + i:4 + i] * xsb[1]
        acc = acc + lf[:, 6 + i:7 + i] * xsb[2]
        fo[...] = acc
    acc = jnp.dot(xb, w1b[...], precision=_HIGH)
    p = pos[...]
    wc = w1c[...]
    acc = acc + p[:, 0:1] * wc[0:1, :]
    acc = acc + p[:, 1:2] * wc[1:2, :]
    acc = acc + p[:, 2:3] * wc[2:3, :]
    g[...] = acc


def _proj_call(x, lf9, pos, w1b, w1c, interpret=False):
    n = x.shape[0]
    grid = (n // _BN,)
    row = lambda i: (i, 0)
    full = lambda i: (0, 0)
    return pl.pallas_call(
        _proj_body,
        grid=grid,
        in_specs=[
            pl.BlockSpec((_BN, _D), row),
            pl.BlockSpec((_BN, 9), row),
            pl.BlockSpec((_BN, 3), row),
            pl.BlockSpec((_D, _H), full),
            pl.BlockSpec((3, _H), full),
        ],
        out_specs=[
            pl.BlockSpec((_BN, _NV), row),
            pl.BlockSpec((_BN, _NV), row),
            pl.BlockSpec((_BN, _NV), row),
            pl.BlockSpec((_BN, _H), row),
        ],
        out_shape=[
            jax.ShapeDtypeStruct((n, _NV), jnp.float32),
            jax.ShapeDtypeStruct((n, _NV), jnp.float32),
            jax.ShapeDtypeStruct((n, _NV), jnp.float32),
            jax.ShapeDtypeStruct((n, _H), jnp.float32),
        ],
        interpret=interpret,
    )(x, lf9, pos, w1b, w1c)


_CHW = 512


def _knn_body(qf0, qf1, qf2, f0, f1, f2, pb, qb, nbr, dist_ref):
    b16 = jnp.bfloat16
    q0, q1, q2 = qf0[...], qf1[...], qf2[...]
    qb_v = qb[...]
    qn = (jnp.sum(q0 * q0, axis=1, keepdims=True)
          + jnp.sum(q1 * q1, axis=1, keepdims=True)
          + jnp.sum(q2 * q2, axis=1, keepdims=True))
    q0b, q1b, q2b = q0.astype(b16), q1.astype(b16), q2.astype(b16)
    pbrow = pb[...]
    lo_q = jnp.zeros_like(qb_v, dtype=jnp.int32)
    hi_q = jnp.zeros_like(qb_v, dtype=jnp.int32)
    for b in range(8):
        fb = jnp.float32(b)
        cnt_lt = jnp.sum((pbrow < fb).astype(jnp.int32), axis=1,
                         keepdims=True)
        cnt_le = jnp.sum((pbrow <= fb).astype(jnp.int32), axis=1,
                         keepdims=True)
        is_b = qb_v == fb
        lo_q = jnp.where(is_b, cnt_lt, lo_q)
        hi_q = jnp.where(is_b, cnt_le, hi_q)
    cnt_q = hi_q - lo_q
    c0 = jnp.min(lo_q) // _CHW
    c1 = (jnp.max(hi_q) + _CHW - 1) // _CHW

    ones = jnp.ones((1, _NV), jnp.float32)
    dn = (((1,), (1,)), ((), ()))

    def fill_body(c, carry):
        off = pl.multiple_of(c * _CHW, _CHW)
        fc0 = f0[pl.ds(off, _CHW), :]
        fc1 = f1[pl.ds(off, _CHW), :]
        fc2 = f2[pl.ds(off, _CHW), :]
        pn_c = (lax.dot_general(ones, fc0 * fc0, dn, precision=_HIGH)
                + lax.dot_general(ones, fc1 * fc1, dn, precision=_HIGH)
                + lax.dot_general(ones, fc2 * fc2, dn, precision=_HIGH))
        dg = (lax.dot_general(q0b, fc0.astype(b16), dn,
                              preferred_element_type=jnp.float32)
              + lax.dot_general(q1b, fc1.astype(b16), dn,
                                preferred_element_type=jnp.float32)
              + lax.dot_general(q2b, fc2.astype(b16), dn,
                                preferred_element_type=jnp.float32))
        d = qn + pn_c - 2.0 * dg
        pbc = pb[:, pl.ds(off, _CHW)]
        d = jnp.where(pbc != qb_v, jnp.float32(1e10), d)
        dist_ref[:, pl.ds(off, _CHW)] = d
        return carry

    lax.fori_loop(c0, c1, fill_body, 0)

    iota_c = lax.broadcasted_iota(jnp.int32, (_BQ, _CHW), 1)
    m_prev = jnp.full((_BQ, 1), -3e38, jnp.float32)
    ix_prev = jnp.full((_BQ, 1), -1, jnp.int32)
    for k in range(_K):
        def scan_body(c, carry):
            m, ix = carry
            off = pl.multiple_of(c * _CHW, _CHW)
            dch = dist_ref[:, pl.ds(off, _CHW)]
            gidx = iota_c + off
            elig = (dch > m_prev) | ((dch == m_prev) & (gidx > ix_prev))
            dsel = jnp.where(elig, dch, jnp.float32(4e38))
            m_c = jnp.min(dsel, axis=1, keepdims=True)
            sel = jnp.where(dsel == m_c, gidx, jnp.int32(2 ** 30))
            ix_c = jnp.min(sel, axis=1, keepdims=True)
            upd = m_c < m
            return (jnp.where(upd, m_c, m), jnp.where(upd, ix_c, ix))

        m_prev, ix_prev = lax.fori_loop(
            c0, c1, scan_body,
            (jnp.full((_BQ, 1), 4e38, jnp.float32),
             jnp.full((_BQ, 1), 2 ** 30, jnp.int32)))

        mq = jnp.int32(k) - cnt_q
        filler = jnp.where(mq < lo_q, mq, hi_q + mq - lo_q)
        nbr[:, k:k + 1] = jnp.where(jnp.int32(k) >= cnt_q, filler, ix_prev)


def _knn_call(qf0, qf1, qf2, f0, f1, f2, pb, qb, interpret=False):
    grid = (_Q // _BQ,)
    row = lambda i: (i, 0)
    full = lambda i: (0, 0)
    return pl.pallas_call(
        _knn_body,
        grid=grid,
        in_specs=[
            pl.BlockSpec((_BQ, _NV), row),
            pl.BlockSpec((_BQ, _NV), row),
            pl.BlockSpec((_BQ, _NV), row),
            pl.BlockSpec((_N, _NV), full),
            pl.BlockSpec((_N, _NV), full),
            pl.BlockSpec((_N, _NV), full),
            pl.BlockSpec((1, _N), full),
            pl.BlockSpec((_BQ, 1), row),
        ],
        out_specs=pl.BlockSpec((_BQ, _K), row),
        out_shape=jax.ShapeDtypeStruct((_Q, _K), jnp.int32),
        scratch_shapes=[pltpu.VMEM((_BQ, _N), jnp.float32)],
        compiler_params=pltpu.CompilerParams(
            vmem_limit_bytes=100 * 1024 * 1024),
        interpret=interpret,
    )(qf0, qf1, qf2, f0, f1, f2, pb, qb)


@functools.cache
def _gather_call():
    nw = 32
    b = _Q * _K
    b_per_w = b // nw
    ch = 128
    mesh = plsc.VectorSubcoreMesh(core_axis_name="c", subcore_axis_name="s")

    @functools.partial(
        pl.kernel, mesh=mesh,
        out_type=jax.ShapeDtypeStruct((b, _H), jnp.float32),
        scratch_types=[
            pltpu.VMEM((ch,), jnp.int32),
            pltpu.VMEM((ch, _H), jnp.float32),
            pltpu.SemaphoreType.DMA,
        ],
    )
    def gather_rows(table_hbm, idx_hbm, out_hbm, idx_v, rows_v, sem):
        wid = lax.axis_index("s") * 2 + lax.axis_index("c")
        base = wid * b_per_w

        def body(i, carry):
            off = base + i * ch
            pltpu.sync_copy(idx_hbm.at[pl.ds(off, ch)], idx_v)
            pltpu.async_copy(table_hbm.at[idx_v], rows_v, sem).wait()
            pltpu.sync_copy(rows_v, out_hbm.at[pl.ds(off, ch)])
            return carry

        lax.fori_loop(0, b_per_w // ch, body, 0)

    return gather_rows


def _mlp_body(gn, qx, pd, wab, w1c, b1, w2, b2, out):
    c = jnp.dot(qx[...], wab[...], precision=_HIGH)
    p = pd[...]
    wc = w1c[...]
    c = c - p[:, 0:1] * wc[0:1, :]
    c = c - p[:, 1:2] * wc[1:2, :]
    c = c - p[:, 2:3] * wc[2:3, :]
    c = c + b1[...]
    h = jnp.maximum(gn[...] + c[:, None, :], 0.0)
    b16 = jnp.bfloat16
    h2 = jnp.dot(h.reshape(_BM * _K, _H).astype(b16), w2[...].astype(b16),
                 preferred_element_type=jnp.float32) + b2[...]
    out[...] = jnp.max(h2.reshape(_BM, _K, _H), axis=1)


def _mlp_call(gn, qx, pd, wab, w1c, b1, w2, b2, interpret=False):
    grid = (_Q // _BM,)
    row = lambda i: (i, 0)
    row3 = lambda i: (i, 0, 0)
    full = lambda i: (0, 0)
    return pl.pallas_call(
        _mlp_body,
        grid=grid,
        in_specs=[
            pl.BlockSpec((_BM, _K, _H), row3),
            pl.BlockSpec((_BM, _D), row),
            pl.BlockSpec((_BM, 3), row),
            pl.BlockSpec((_D, _H), full),
            pl.BlockSpec((3, _H), full),
            pl.BlockSpec((1, _H), full),
            pl.BlockSpec((_H, _H), full),
            pl.BlockSpec((1, _H), full),
        ],
        out_specs=pl.BlockSpec((_BM, _H), row),
        out_shape=jax.ShapeDtypeStruct((_Q, _H), jnp.float32),
        interpret=interpret,
    )(gn, qx, pd, wab, w1c, b1, w2, b2)


def kernel(x, pos, batch, lframes, W1, b1, W2, b2):
    n, d = x.shape
    lf9 = lframes.reshape(n, 9)

    W1a = W1[:d]
    W1b = W1[d:2 * d]
    w1c = W1[2 * d:]

    f0, f1, f2, g = _proj_call(x, lf9, pos, W1b, w1c)

    bf = batch.astype(jnp.float32)
    nbr = _knn_call(
        f0[::_STRIDE], f1[::_STRIDE], f2[::_STRIDE],
        f0, f1, f2,
        bf.reshape(1, n), bf[::_STRIDE].reshape(_Q, 1))

    gn = _gather_call()(g, nbr.reshape(-1))
    gn = gn.reshape(_Q, _K, _H)

    out = _mlp_call(
        gn, x[::_STRIDE], pos[::_STRIDE], W1a - W1b,
        w1c, b1.reshape(1, _H), W2, b2.reshape(1, _H))

    return (out, pos[::_STRIDE], batch[::_STRIDE], lframes[::_STRIDE])


# speedup vs baseline: 1.0241x; 1.0241x over previous
import functools

import jax
import jax.numpy as jnp
from jax import lax
from jax.experimental import pallas as pl
from jax.experimental.pallas import tpu as pltpu
from jax.experimental.pallas import tpu_sc as plsc

_N = 16384
_D = 192
_H = 384
_K = 16
_STRIDE = 4
_Q = _N // _STRIDE
_NV = _D // 3
_BN = 2048
_BQ = 128
_BM = 128
_HIGH = lax.Precision.HIGHEST


def _proj_body(xr, lf9, pos, w1b, w1c, f0, f1, f2, g):
    xb = xr[...]
    rows = lax.broadcasted_iota(jnp.int32, (_D, _NV), 0)
    cols = lax.broadcasted_iota(jnp.int32, (_D, _NV), 1)
    xs = tuple(
        jnp.dot(xb, (rows == 3 * cols + j).astype(jnp.float32),
                precision=_HIGH)
        for j in range(3))
    xsb = tuple(v.astype(jnp.bfloat16).astype(jnp.float32) for v in xs)
    lf = lf9[...].astype(jnp.bfloat16).astype(jnp.float32)
    for i, fo in enumerate((f0, f1, f2)):
        acc = lf[:, i:i + 1] * xsb[0]
        acc = acc + lf[:, 3 + i:4 + i] * xsb[1]
        acc = acc + lf[:, 6 + i:7 + i] * xsb[2]
        fo[...] = acc
    acc = jnp.dot(xb, w1b[...], precision=_HIGH)
    p = pos[...]
    wc = w1c[...]
    acc = acc + p[:, 0:1] * wc[0:1, :]
    acc = acc + p[:, 1:2] * wc[1:2, :]
    acc = acc + p[:, 2:3] * wc[2:3, :]
    g[...] = acc


def _proj_call(x, lf9, pos, w1b, w1c, interpret=False):
    n = x.shape[0]
    grid = (n // _BN,)
    row = lambda i: (i, 0)
    full = lambda i: (0, 0)
    return pl.pallas_call(
        _proj_body,
        grid=grid,
        in_specs=[
            pl.BlockSpec((_BN, _D), row),
            pl.BlockSpec((_BN, 9), row),
            pl.BlockSpec((_BN, 3), row),
            pl.BlockSpec((_D, _H), full),
            pl.BlockSpec((3, _H), full),
        ],
        out_specs=[
            pl.BlockSpec((_BN, _NV), row),
            pl.BlockSpec((_BN, _NV), row),
            pl.BlockSpec((_BN, _NV), row),
            pl.BlockSpec((_BN, _H), row),
        ],
        out_shape=[
            jax.ShapeDtypeStruct((n, _NV), jnp.float32),
            jax.ShapeDtypeStruct((n, _NV), jnp.float32),
            jax.ShapeDtypeStruct((n, _NV), jnp.float32),
            jax.ShapeDtypeStruct((n, _H), jnp.float32),
        ],
        interpret=interpret,
    )(x, lf9, pos, w1b, w1c)


_CHW = 512


def _knn_body(qf0, qf1, qf2, f0, f1, f2, pb, qb, nbr, dist_ref):
    b16 = jnp.bfloat16
    q0, q1, q2 = qf0[...], qf1[...], qf2[...]
    qb_v = qb[...]
    qn = (jnp.sum(q0 * q0, axis=1, keepdims=True)
          + jnp.sum(q1 * q1, axis=1, keepdims=True)
          + jnp.sum(q2 * q2, axis=1, keepdims=True))
    q0b, q1b, q2b = q0.astype(b16), q1.astype(b16), q2.astype(b16)
    pbrow = pb[...]
    lo_q = jnp.zeros_like(qb_v, dtype=jnp.int32)
    hi_q = jnp.zeros_like(qb_v, dtype=jnp.int32)
    for b in range(8):
        fb = jnp.float32(b)
        cnt_lt = jnp.sum((pbrow < fb).astype(jnp.int32), axis=1,
                         keepdims=True)
        cnt_le = jnp.sum((pbrow <= fb).astype(jnp.int32), axis=1,
                         keepdims=True)
        is_b = qb_v == fb
        lo_q = jnp.where(is_b, cnt_lt, lo_q)
        hi_q = jnp.where(is_b, cnt_le, hi_q)
    cnt_q = hi_q - lo_q
    c0 = jnp.min(lo_q) // _CHW
    c1 = (jnp.max(hi_q) + _CHW - 1) // _CHW

    ones = jnp.ones((1, _NV), jnp.float32)
    dn = (((1,), (1,)), ((), ()))

    def fill_body(c, carry):
        off = pl.multiple_of(c * _CHW, _CHW)
        fc0 = f0[pl.ds(off, _CHW), :]
        fc1 = f1[pl.ds(off, _CHW), :]
        fc2 = f2[pl.ds(off, _CHW), :]
        pn_c = (lax.dot_general(ones, fc0 * fc0, dn, precision=_HIGH)
                + lax.dot_general(ones, fc1 * fc1, dn, precision=_HIGH)
                + lax.dot_general(ones, fc2 * fc2, dn, precision=_HIGH))
        dg = (lax.dot_general(q0b, fc0.astype(b16), dn,
                              preferred_element_type=jnp.float32)
              + lax.dot_general(q1b, fc1.astype(b16), dn,
                                preferred_element_type=jnp.float32)
              + lax.dot_general(q2b, fc2.astype(b16), dn,
                                preferred_element_type=jnp.float32))
        d = qn + pn_c - 2.0 * dg
        pbc = pb[:, pl.ds(off, _CHW)]
        d = jnp.where(pbc != qb_v, jnp.float32(1e10), d)
        dist_ref[:, pl.ds(off, _CHW)] = d
        return carry

    lax.fori_loop(c0, c1, fill_body, 0)

    iota_c = lax.broadcasted_iota(jnp.int32, (_BQ, _CHW), 1)
    for k in range(_K):
        def scan_body(c, carry):
            m, ix = carry
            off = pl.multiple_of(c * _CHW, _CHW)
            dch = dist_ref[:, pl.ds(off, _CHW)]
            m_c = jnp.min(dch, axis=1, keepdims=True)
            sel = jnp.where(dch == m_c, iota_c, jnp.int32(2 ** 30))
            ix_c = jnp.min(sel, axis=1, keepdims=True) + off
            upd = m_c < m
            return (jnp.where(upd, m_c, m), jnp.where(upd, ix_c, ix))

        m, ix = lax.fori_loop(
            c0, c1, scan_body,
            (jnp.full((_BQ, 1), 4e10, jnp.float32),
             jnp.full((_BQ, 1), 2 ** 30, jnp.int32)))

        def mask_body(c, carry):
            off = pl.multiple_of(c * _CHW, _CHW)
            dch = dist_ref[:, pl.ds(off, _CHW)]
            dist_ref[:, pl.ds(off, _CHW)] = jnp.where(
                iota_c + off == ix, jnp.float32(3e10), dch)
            return carry

        lax.fori_loop(c0, c1, mask_body, 0)

        mq = jnp.int32(k) - cnt_q
        filler = jnp.where(mq < lo_q, mq, hi_q + mq - lo_q)
        nbr[:, k:k + 1] = jnp.where(jnp.int32(k) >= cnt_q, filler, ix)


def _knn_call(qf0, qf1, qf2, f0, f1, f2, pb, qb, interpret=False):
    grid = (_Q // _BQ,)
    row = lambda i: (i, 0)
    full = lambda i: (0, 0)
    return pl.pallas_call(
        _knn_body,
        grid=grid,
        in_specs=[
            pl.BlockSpec((_BQ, _NV), row),
            pl.BlockSpec((_BQ, _NV), row),
            pl.BlockSpec((_BQ, _NV), row),
            pl.BlockSpec((_N, _NV), full),
            pl.BlockSpec((_N, _NV), full),
            pl.BlockSpec((_N, _NV), full),
            pl.BlockSpec((1, _N), full),
            pl.BlockSpec((_BQ, 1), row),
        ],
        out_specs=pl.BlockSpec((_BQ, _K), row),
        out_shape=jax.ShapeDtypeStruct((_Q, _K), jnp.int32),
        scratch_shapes=[pltpu.VMEM((_BQ, _N), jnp.float32)],
        compiler_params=pltpu.CompilerParams(
            vmem_limit_bytes=100 * 1024 * 1024),
        interpret=interpret,
    )(qf0, qf1, qf2, f0, f1, f2, pb, qb)


@functools.cache
def _gather_call():
    nw = 32
    b = _Q * _K
    b_per_w = b // nw
    ch = 128
    mesh = plsc.VectorSubcoreMesh(core_axis_name="c", subcore_axis_name="s")

    @functools.partial(
        pl.kernel, mesh=mesh,
        out_type=jax.ShapeDtypeStruct((b, _H), jnp.float32),
        scratch_types=[
            pltpu.VMEM((ch,), jnp.int32),
            pltpu.VMEM((ch, _H), jnp.float32),
            pltpu.SemaphoreType.DMA,
        ],
    )
    def gather_rows(table_hbm, idx_hbm, out_hbm, idx_v, rows_v, sem):
        wid = lax.axis_index("s") * 2 + lax.axis_index("c")
        base = wid * b_per_w

        def body(i, carry):
            off = base + i * ch
            pltpu.sync_copy(idx_hbm.at[pl.ds(off, ch)], idx_v)
            pltpu.async_copy(table_hbm.at[idx_v], rows_v, sem).wait()
            pltpu.sync_copy(rows_v, out_hbm.at[pl.ds(off, ch)])
            return carry

        lax.fori_loop(0, b_per_w // ch, body, 0)

    return gather_rows


def _mlp_body(gn, qx, pd, wab, w1c, b1, w2, b2, out):
    c = jnp.dot(qx[...], wab[...], precision=_HIGH)
    p = pd[...]
    wc = w1c[...]
    c = c - p[:, 0:1] * wc[0:1, :]
    c = c - p[:, 1:2] * wc[1:2, :]
    c = c - p[:, 2:3] * wc[2:3, :]
    c = c + b1[...]
    h = jnp.maximum(gn[...] + c[:, None, :], 0.0)
    b16 = jnp.bfloat16
    h2 = jnp.dot(h.reshape(_BM * _K, _H).astype(b16), w2[...].astype(b16),
                 preferred_element_type=jnp.float32) + b2[...]
    out[...] = jnp.max(h2.reshape(_BM, _K, _H), axis=1)


def _mlp_call(gn, qx, pd, wab, w1c, b1, w2, b2, interpret=False):
    grid = (_Q // _BM,)
    row = lambda i: (i, 0)
    row3 = lambda i: (i, 0, 0)
    full = lambda i: (0, 0)
    return pl.pallas_call(
        _mlp_body,
        grid=grid,
        in_specs=[
            pl.BlockSpec((_BM, _K, _H), row3),
            pl.BlockSpec((_BM, _D), row),
            pl.BlockSpec((_BM, 3), row),
            pl.BlockSpec((_D, _H), full),
            pl.BlockSpec((3, _H), full),
            pl.BlockSpec((1, _H), full),
            pl.BlockSpec((_H, _H), full),
            pl.BlockSpec((1, _H), full),
        ],
        out_specs=pl.BlockSpec((_BM, _H), row),
        out_shape=jax.ShapeDtypeStruct((_Q, _H), jnp.float32),
        interpret=interpret,
    )(gn, qx, pd, wab, w1c, b1, w2, b2)


def kernel(x, pos, batch, lframes, W1, b1, W2, b2):
    n, d = x.shape
    lf9 = lframes.reshape(n, 9)

    W1a = W1[:d]
    W1b = W1[d:2 * d]
    w1c = W1[2 * d:]

    f0, f1, f2, g = _proj_call(x, lf9, pos, W1b, w1c)

    bf = batch.astype(jnp.float32)
    nbr = _knn_call(
        f0[::_STRIDE], f1[::_STRIDE], f2[::_STRIDE],
        f0, f1, f2,
        bf.reshape(1, n), bf[::_STRIDE].reshape(_Q, 1))

    gn = _gather_call()(g, nbr.reshape(-1))
    gn = gn.reshape(_Q, _K, _H)

    out = _mlp_call(
        gn, x[::_STRIDE], pos[::_STRIDE], W1a - W1b,
        w1c, b1.reshape(1, _H), W2, b2.reshape(1, _H))

    return (out, pos[::_STRIDE], batch[::_STRIDE], lframes[::_STRIDE])


# speedup vs baseline: 1.2614x; 1.2317x over previous
import functools

import jax
import jax.numpy as jnp
from jax import lax
from jax.experimental import pallas as pl
from jax.experimental.pallas import tpu as pltpu
from jax.experimental.pallas import tpu_sc as plsc

_N = 16384
_D = 192
_H = 384
_K = 16
_STRIDE = 4
_Q = _N // _STRIDE
_NV = _D // 3
_BN = 2048
_BQ = 128
_BM = 128
_HIGH = lax.Precision.HIGHEST


def _proj_body(xr, lf9, pos, w1b, w1c, f0, f1, f2, g):
    xb = xr[...]
    rows = lax.broadcasted_iota(jnp.int32, (_D, _NV), 0)
    cols = lax.broadcasted_iota(jnp.int32, (_D, _NV), 1)
    xs = tuple(
        jnp.dot(xb, (rows == 3 * cols + j).astype(jnp.float32),
                precision=_HIGH)
        for j in range(3))
    xsb = tuple(v.astype(jnp.bfloat16).astype(jnp.float32) for v in xs)
    lf = lf9[...].astype(jnp.bfloat16).astype(jnp.float32)
    for i, fo in enumerate((f0, f1, f2)):
        acc = lf[:, i:i + 1] * xsb[0]
        acc = acc + lf[:, 3 + i:4 + i] * xsb[1]
        acc = acc + lf[:, 6 + i:7 + i] * xsb[2]
        fo[...] = acc
    acc = jnp.dot(xb, w1b[...], precision=_HIGH)
    p = pos[...]
    wc = w1c[...]
    acc = acc + p[:, 0:1] * wc[0:1, :]
    acc = acc + p[:, 1:2] * wc[1:2, :]
    acc = acc + p[:, 2:3] * wc[2:3, :]
    g[...] = acc


def _proj_call(x, lf9, pos, w1b, w1c, interpret=False):
    n = x.shape[0]
    grid = (n // _BN,)
    row = lambda i: (i, 0)
    full = lambda i: (0, 0)
    return pl.pallas_call(
        _proj_body,
        grid=grid,
        in_specs=[
            pl.BlockSpec((_BN, _D), row),
            pl.BlockSpec((_BN, 9), row),
            pl.BlockSpec((_BN, 3), row),
            pl.BlockSpec((_D, _H), full),
            pl.BlockSpec((3, _H), full),
        ],
        out_specs=[
            pl.BlockSpec((_BN, _NV), row),
            pl.BlockSpec((_BN, _NV), row),
            pl.BlockSpec((_BN, _NV), row),
            pl.BlockSpec((_BN, _H), row),
        ],
        out_shape=[
            jax.ShapeDtypeStruct((n, _NV), jnp.float32),
            jax.ShapeDtypeStruct((n, _NV), jnp.float32),
            jax.ShapeDtypeStruct((n, _NV), jnp.float32),
            jax.ShapeDtypeStruct((n, _H), jnp.float32),
        ],
        interpret=interpret,
    )(x, lf9, pos, w1b, w1c)


_CHW = 1024


def _knn_body(qf0, qf1, qf2, f0, f1, f2, pb, qb, nbr, dist_ref):
    b16 = jnp.bfloat16
    q0, q1, q2 = qf0[...], qf1[...], qf2[...]
    qb_v = qb[...]
    qn = (jnp.sum(q0 * q0, axis=1, keepdims=True)
          + jnp.sum(q1 * q1, axis=1, keepdims=True)
          + jnp.sum(q2 * q2, axis=1, keepdims=True))
    q0b, q1b, q2b = q0.astype(b16), q1.astype(b16), q2.astype(b16)
    pbrow = pb[...]
    lo_q = jnp.sum((pbrow < qb_v).astype(jnp.int32), axis=1, keepdims=True)
    hi_q = jnp.sum((pbrow <= qb_v).astype(jnp.int32), axis=1, keepdims=True)
    cnt_q = hi_q - lo_q
    c0 = jnp.min(lo_q) // _CHW
    c1 = (jnp.max(hi_q) + _CHW - 1) // _CHW

    ones = jnp.ones((1, _NV), jnp.float32)
    dn = (((1,), (1,)), ((), ()))

    def fill_body(c, carry):
        off = pl.multiple_of(c * _CHW, _CHW)
        fc0 = f0[pl.ds(off, _CHW), :]
        fc1 = f1[pl.ds(off, _CHW), :]
        fc2 = f2[pl.ds(off, _CHW), :]
        pn_c = (lax.dot_general(ones, fc0 * fc0, dn, precision=_HIGH)
                + lax.dot_general(ones, fc1 * fc1, dn, precision=_HIGH)
                + lax.dot_general(ones, fc2 * fc2, dn, precision=_HIGH))
        dg = (lax.dot_general(q0b, fc0.astype(b16), dn,
                              preferred_element_type=jnp.float32)
              + lax.dot_general(q1b, fc1.astype(b16), dn,
                                preferred_element_type=jnp.float32)
              + lax.dot_general(q2b, fc2.astype(b16), dn,
                                preferred_element_type=jnp.float32))
        d = qn + pn_c - 2.0 * dg
        pbc = pb[:, pl.ds(off, _CHW)]
        d = jnp.where(pbc != qb_v, jnp.float32(1e10), d)
        dist_ref[:, pl.ds(off, _CHW)] = d
        return carry

    lax.fori_loop(c0, c1, fill_body, 0)

    iota_c = lax.broadcasted_iota(jnp.int32, (_BQ, _CHW), 1)
    for k in range(_K):
        def scan_body(c, carry):
            m, ix = carry
            off = pl.multiple_of(c * _CHW, _CHW)
            dch = dist_ref[:, pl.ds(off, _CHW)]
            m_c = jnp.min(dch, axis=1, keepdims=True)
            sel = jnp.where(dch == m_c, iota_c, jnp.int32(2 ** 30))
            ix_c = jnp.min(sel, axis=1, keepdims=True) + off
            upd = m_c < m
            return (jnp.where(upd, m_c, m), jnp.where(upd, ix_c, ix))

        m, ix = lax.fori_loop(
            c0, c1, scan_body,
            (jnp.full((_BQ, 1), 4e10, jnp.float32),
             jnp.full((_BQ, 1), 2 ** 30, jnp.int32)))

        def mask_body(c, carry):
            off = pl.multiple_of(c * _CHW, _CHW)
            dch = dist_ref[:, pl.ds(off, _CHW)]
            dist_ref[:, pl.ds(off, _CHW)] = jnp.where(
                iota_c + off == ix, jnp.float32(3e10), dch)
            return carry

        if k < _K - 1:
            lax.fori_loop(c0, c1, mask_body, 0)

        mq = jnp.int32(k) - cnt_q
        filler = jnp.where(mq < lo_q, mq, hi_q + mq - lo_q)
        nbr[:, k:k + 1] = jnp.where(jnp.int32(k) >= cnt_q, filler, ix)


def _knn_call(qf0, qf1, qf2, f0, f1, f2, pb, qb, interpret=False):
    grid = (_Q // _BQ,)
    row = lambda i: (i, 0)
    full = lambda i: (0, 0)
    return pl.pallas_call(
        _knn_body,
        grid=grid,
        in_specs=[
            pl.BlockSpec((_BQ, _NV), row),
            pl.BlockSpec((_BQ, _NV), row),
            pl.BlockSpec((_BQ, _NV), row),
            pl.BlockSpec((_N, _NV), full),
            pl.BlockSpec((_N, _NV), full),
            pl.BlockSpec((_N, _NV), full),
            pl.BlockSpec((1, _N), full),
            pl.BlockSpec((_BQ, 1), row),
        ],
        out_specs=pl.BlockSpec((_BQ, _K), row),
        out_shape=jax.ShapeDtypeStruct((_Q, _K), jnp.int32),
        scratch_shapes=[pltpu.VMEM((_BQ, _N), jnp.float32)],
        compiler_params=pltpu.CompilerParams(
            vmem_limit_bytes=100 * 1024 * 1024),
        interpret=interpret,
    )(qf0, qf1, qf2, f0, f1, f2, pb, qb)


@functools.cache
def _gather_call():
    nw = 32
    b = _Q * _K
    b_per_w = b // nw
    ch = 128
    mesh = plsc.VectorSubcoreMesh(core_axis_name="c", subcore_axis_name="s")

    @functools.partial(
        pl.kernel, mesh=mesh,
        out_type=jax.ShapeDtypeStruct((b, _H), jnp.float32),
        scratch_types=[
            pltpu.VMEM((ch,), jnp.int32),
            pltpu.VMEM((ch, _H), jnp.float32),
            pltpu.SemaphoreType.DMA,
        ],
    )
    def gather_rows(table_hbm, idx_hbm, out_hbm, idx_v, rows_v, sem):
        wid = lax.axis_index("s") * 2 + lax.axis_index("c")
        base = wid * b_per_w

        def body(i, carry):
            off = base + i * ch
            pltpu.sync_copy(idx_hbm.at[pl.ds(off, ch)], idx_v)
            pltpu.async_copy(table_hbm.at[idx_v], rows_v, sem).wait()
            pltpu.sync_copy(rows_v, out_hbm.at[pl.ds(off, ch)])
            return carry

        lax.fori_loop(0, b_per_w // ch, body, 0)

    return gather_rows


def _mlp_body(gn, qx, pd, wab, w1c, b1, w2, b2, out):
    c = jnp.dot(qx[...], wab[...], precision=_HIGH)
    p = pd[...]
    wc = w1c[...]
    c = c - p[:, 0:1] * wc[0:1, :]
    c = c - p[:, 1:2] * wc[1:2, :]
    c = c - p[:, 2:3] * wc[2:3, :]
    c = c + b1[...]
    h = jnp.maximum(gn[...] + c[:, None, :], 0.0)
    b16 = jnp.bfloat16
    h2 = jnp.dot(h.reshape(_BM * _K, _H).astype(b16), w2[...].astype(b16),
                 preferred_element_type=jnp.float32) + b2[...]
    out[...] = jnp.max(h2.reshape(_BM, _K, _H), axis=1)


def _mlp_call(gn, qx, pd, wab, w1c, b1, w2, b2, interpret=False):
    grid = (_Q // _BM,)
    row = lambda i: (i, 0)
    row3 = lambda i: (i, 0, 0)
    full = lambda i: (0, 0)
    return pl.pallas_call(
        _mlp_body,
        grid=grid,
        in_specs=[
            pl.BlockSpec((_BM, _K, _H), row3),
            pl.BlockSpec((_BM, _D), row),
            pl.BlockSpec((_BM, 3), row),
            pl.BlockSpec((_D, _H), full),
            pl.BlockSpec((3, _H), full),
            pl.BlockSpec((1, _H), full),
            pl.BlockSpec((_H, _H), full),
            pl.BlockSpec((1, _H), full),
        ],
        out_specs=pl.BlockSpec((_BM, _H), row),
        out_shape=jax.ShapeDtypeStruct((_Q, _H), jnp.float32),
        interpret=interpret,
    )(gn, qx, pd, wab, w1c, b1, w2, b2)


def kernel(x, pos, batch, lframes, W1, b1, W2, b2):
    n, d = x.shape
    lf9 = lframes.reshape(n, 9)

    W1a = W1[:d]
    W1b = W1[d:2 * d]
    w1c = W1[2 * d:]

    f0, f1, f2, g = _proj_call(x, lf9, pos, W1b, w1c)

    bf = batch.astype(jnp.float32)
    nbr = _knn_call(
        f0[::_STRIDE], f1[::_STRIDE], f2[::_STRIDE],
        f0, f1, f2,
        bf.reshape(1, n), bf[::_STRIDE].reshape(_Q, 1))

    gn = _gather_call()(g, nbr.reshape(-1))
    gn = gn.reshape(_Q, _K, _H)

    out = _mlp_call(
        gn, x[::_STRIDE], pos[::_STRIDE], W1a - W1b,
        w1c, b1.reshape(1, _H), W2, b2.reshape(1, _H))

    return (out, pos[::_STRIDE], batch[::_STRIDE], lframes[::_STRIDE])


# speedup vs baseline: 1.3274x; 1.0524x over previous
import functools

import jax
import jax.numpy as jnp
from jax import lax
from jax.experimental import pallas as pl
from jax.experimental.pallas import tpu as pltpu
from jax.experimental.pallas import tpu_sc as plsc

_N = 16384
_D = 192
_H = 384
_K = 16
_STRIDE = 4
_Q = _N // _STRIDE
_NV = _D // 3
_BN = 2048
_BQ = 128
_BM = 128
_HIGH = lax.Precision.HIGHEST


def _proj_body(xr, lf9, pos, w1b, w1c, f0, f1, f2, g):
    xb = xr[...]
    rows = lax.broadcasted_iota(jnp.int32, (_D, _NV), 0)
    cols = lax.broadcasted_iota(jnp.int32, (_D, _NV), 1)
    xs = tuple(
        jnp.dot(xb, (rows == 3 * cols + j).astype(jnp.float32),
                precision=_HIGH)
        for j in range(3))
    xsb = tuple(v.astype(jnp.bfloat16).astype(jnp.float32) for v in xs)
    lf = lf9[...].astype(jnp.bfloat16).astype(jnp.float32)
    for i, fo in enumerate((f0, f1, f2)):
        acc = lf[:, i:i + 1] * xsb[0]
        acc = acc + lf[:, 3 + i:4 + i] * xsb[1]
        acc = acc + lf[:, 6 + i:7 + i] * xsb[2]
        fo[...] = acc
    acc = jnp.dot(xb, w1b[...], precision=_HIGH)
    p = pos[...]
    wc = w1c[...]
    acc = acc + p[:, 0:1] * wc[0:1, :]
    acc = acc + p[:, 1:2] * wc[1:2, :]
    acc = acc + p[:, 2:3] * wc[2:3, :]
    g[...] = acc


def _proj_call(x, lf9, pos, w1b, w1c, interpret=False):
    n = x.shape[0]
    grid = (n // _BN,)
    row = lambda i: (i, 0)
    full = lambda i: (0, 0)
    return pl.pallas_call(
        _proj_body,
        grid=grid,
        in_specs=[
            pl.BlockSpec((_BN, _D), row),
            pl.BlockSpec((_BN, 9), row),
            pl.BlockSpec((_BN, 3), row),
            pl.BlockSpec((_D, _H), full),
            pl.BlockSpec((3, _H), full),
        ],
        out_specs=[
            pl.BlockSpec((_BN, _NV), row),
            pl.BlockSpec((_BN, _NV), row),
            pl.BlockSpec((_BN, _NV), row),
            pl.BlockSpec((_BN, _H), row),
        ],
        out_shape=[
            jax.ShapeDtypeStruct((n, _NV), jnp.float32),
            jax.ShapeDtypeStruct((n, _NV), jnp.float32),
            jax.ShapeDtypeStruct((n, _NV), jnp.float32),
            jax.ShapeDtypeStruct((n, _H), jnp.float32),
        ],
        interpret=interpret,
    )(x, lf9, pos, w1b, w1c)


_CHW = 2048


def _knn_body(qf0, qf1, qf2, f0, f1, f2, pb, qb, nbr, dist_ref):
    b16 = jnp.bfloat16
    q0, q1, q2 = qf0[...], qf1[...], qf2[...]
    qb_v = qb[...]
    qn = (jnp.sum(q0 * q0, axis=1, keepdims=True)
          + jnp.sum(q1 * q1, axis=1, keepdims=True)
          + jnp.sum(q2 * q2, axis=1, keepdims=True))
    q0b, q1b, q2b = q0.astype(b16), q1.astype(b16), q2.astype(b16)
    pbrow = pb[...]
    lo_q = jnp.sum((pbrow < qb_v).astype(jnp.int32), axis=1, keepdims=True)
    hi_q = jnp.sum((pbrow <= qb_v).astype(jnp.int32), axis=1, keepdims=True)
    cnt_q = hi_q - lo_q
    c0 = jnp.min(lo_q) // _CHW
    c1 = (jnp.max(hi_q) + _CHW - 1) // _CHW

    ones = jnp.ones((1, _NV), jnp.float32)
    dn = (((1,), (1,)), ((), ()))

    def fill_body(c, carry):
        off = pl.multiple_of(c * _CHW, _CHW)
        fc0 = f0[pl.ds(off, _CHW), :]
        fc1 = f1[pl.ds(off, _CHW), :]
        fc2 = f2[pl.ds(off, _CHW), :]
        pn_c = (lax.dot_general(ones, fc0 * fc0, dn, precision=_HIGH)
                + lax.dot_general(ones, fc1 * fc1, dn, precision=_HIGH)
                + lax.dot_general(ones, fc2 * fc2, dn, precision=_HIGH))
        dg = (lax.dot_general(q0b, fc0.astype(b16), dn,
                              preferred_element_type=jnp.float32)
              + lax.dot_general(q1b, fc1.astype(b16), dn,
                                preferred_element_type=jnp.float32)
              + lax.dot_general(q2b, fc2.astype(b16), dn,
                                preferred_element_type=jnp.float32))
        d = qn + pn_c - 2.0 * dg
        pbc = pb[:, pl.ds(off, _CHW)]
        d = jnp.where(pbc != qb_v, jnp.float32(1e10), d)
        dist_ref[:, pl.ds(off, _CHW)] = d
        return carry

    lax.fori_loop(c0, c1, fill_body, 0)

    iota_c = lax.broadcasted_iota(jnp.int32, (_BQ, _CHW), 1)
    for k in range(_K):
        def scan_body(c, carry):
            m, ix = carry
            off = pl.multiple_of(c * _CHW, _CHW)
            dch = dist_ref[:, pl.ds(off, _CHW)]
            m_c = jnp.min(dch, axis=1, keepdims=True)
            sel = jnp.where(dch == m_c, iota_c, jnp.int32(2 ** 30))
            ix_c = jnp.min(sel, axis=1, keepdims=True) + off
            upd = m_c < m
            return (jnp.where(upd, m_c, m), jnp.where(upd, ix_c, ix))

        m, ix = lax.fori_loop(
            c0, c1, scan_body,
            (jnp.full((_BQ, 1), 4e10, jnp.float32),
             jnp.full((_BQ, 1), 2 ** 30, jnp.int32)))

        def mask_body(c, carry):
            off = pl.multiple_of(c * _CHW, _CHW)
            dch = dist_ref[:, pl.ds(off, _CHW)]
            dist_ref[:, pl.ds(off, _CHW)] = jnp.where(
                iota_c + off == ix, jnp.float32(3e10), dch)
            return carry

        if k < _K - 1:
            lax.fori_loop(c0, c1, mask_body, 0)

        mq = jnp.int32(k) - cnt_q
        filler = jnp.where(mq < lo_q, mq, hi_q + mq - lo_q)
        nbr[:, k:k + 1] = jnp.where(jnp.int32(k) >= cnt_q, filler, ix)


def _knn_call(qf0, qf1, qf2, f0, f1, f2, pb, qb, interpret=False):
    grid = (_Q // _BQ,)
    row = lambda i: (i, 0)
    full = lambda i: (0, 0)
    return pl.pallas_call(
        _knn_body,
        grid=grid,
        in_specs=[
            pl.BlockSpec((_BQ, _NV), row),
            pl.BlockSpec((_BQ, _NV), row),
            pl.BlockSpec((_BQ, _NV), row),
            pl.BlockSpec((_N, _NV), full),
            pl.BlockSpec((_N, _NV), full),
            pl.BlockSpec((_N, _NV), full),
            pl.BlockSpec((1, _N), full),
            pl.BlockSpec((_BQ, 1), row),
        ],
        out_specs=pl.BlockSpec((_BQ, _K), row),
        out_shape=jax.ShapeDtypeStruct((_Q, _K), jnp.int32),
        scratch_shapes=[pltpu.VMEM((_BQ, _N), jnp.float32)],
        compiler_params=pltpu.CompilerParams(
            vmem_limit_bytes=100 * 1024 * 1024),
        interpret=interpret,
    )(qf0, qf1, qf2, f0, f1, f2, pb, qb)


@functools.cache
def _gather_call():
    nw = 32
    b = _Q * _K
    b_per_w = b // nw
    ch = 128
    mesh = plsc.VectorSubcoreMesh(core_axis_name="c", subcore_axis_name="s")

    @functools.partial(
        pl.kernel, mesh=mesh,
        out_type=jax.ShapeDtypeStruct((b, _H), jnp.float32),
        scratch_types=[
            pltpu.VMEM((ch,), jnp.int32),
            pltpu.VMEM((ch, _H), jnp.float32),
            pltpu.SemaphoreType.DMA,
        ],
    )
    def gather_rows(table_hbm, idx_hbm, out_hbm, idx_v, rows_v, sem):
        wid = lax.axis_index("s") * 2 + lax.axis_index("c")
        base = wid * b_per_w

        def body(i, carry):
            off = base + i * ch
            pltpu.sync_copy(idx_hbm.at[pl.ds(off, ch)], idx_v)
            pltpu.async_copy(table_hbm.at[idx_v], rows_v, sem).wait()
            pltpu.sync_copy(rows_v, out_hbm.at[pl.ds(off, ch)])
            return carry

        lax.fori_loop(0, b_per_w // ch, body, 0)

    return gather_rows


def _mlp_body(gn, qx, pd, wab, w1c, b1, w2, b2, out):
    c = jnp.dot(qx[...], wab[...], precision=_HIGH)
    p = pd[...]
    wc = w1c[...]
    c = c - p[:, 0:1] * wc[0:1, :]
    c = c - p[:, 1:2] * wc[1:2, :]
    c = c - p[:, 2:3] * wc[2:3, :]
    c = c + b1[...]
    h = jnp.maximum(gn[...] + c[:, None, :], 0.0)
    b16 = jnp.bfloat16
    h2 = jnp.dot(h.reshape(_BM * _K, _H).astype(b16), w2[...].astype(b16),
                 preferred_element_type=jnp.float32) + b2[...]
    out[...] = jnp.max(h2.reshape(_BM, _K, _H), axis=1)


def _mlp_call(gn, qx, pd, wab, w1c, b1, w2, b2, interpret=False):
    grid = (_Q // _BM,)
    row = lambda i: (i, 0)
    row3 = lambda i: (i, 0, 0)
    full = lambda i: (0, 0)
    return pl.pallas_call(
        _mlp_body,
        grid=grid,
        in_specs=[
            pl.BlockSpec((_BM, _K, _H), row3),
            pl.BlockSpec((_BM, _D), row),
            pl.BlockSpec((_BM, 3), row),
            pl.BlockSpec((_D, _H), full),
            pl.BlockSpec((3, _H), full),
            pl.BlockSpec((1, _H), full),
            pl.BlockSpec((_H, _H), full),
            pl.BlockSpec((1, _H), full),
        ],
        out_specs=pl.BlockSpec((_BM, _H), row),
        out_shape=jax.ShapeDtypeStruct((_Q, _H), jnp.float32),
        interpret=interpret,
    )(gn, qx, pd, wab, w1c, b1, w2, b2)


def kernel(x, pos, batch, lframes, W1, b1, W2, b2):
    n, d = x.shape
    lf9 = lframes.reshape(n, 9)

    W1a = W1[:d]
    W1b = W1[d:2 * d]
    w1c = W1[2 * d:]

    f0, f1, f2, g = _proj_call(x, lf9, pos, W1b, w1c)

    bf = batch.astype(jnp.float32)
    nbr = _knn_call(
        f0[::_STRIDE], f1[::_STRIDE], f2[::_STRIDE],
        f0, f1, f2,
        bf.reshape(1, n), bf[::_STRIDE].reshape(_Q, 1))

    gn = _gather_call()(g, nbr.reshape(-1))
    gn = gn.reshape(_Q, _K, _H)

    out = _mlp_call(
        gn, x[::_STRIDE], pos[::_STRIDE], W1a - W1b,
        w1c, b1.reshape(1, _H), W2, b2.reshape(1, _H))

    return (out, pos[::_STRIDE], batch[::_STRIDE], lframes[::_STRIDE])
